# baseline (speedup 1.0000x reference)
import jax
import jax.numpy as jnp
from jax.experimental import pallas as pl
from jax.experimental.pallas import tpu as pltpu

_LANE = 128
_TN = 8192


def _round_up(v, m):
    return (v + m - 1) // m * m


def _maha_kernel(x_ref, means_ref, alpha_ref, out_ref,
                 abb_ref, m2_ref, t4_ref):
    bf16 = jnp.bfloat16
    f32 = jnp.float32

    @pl.when(pl.program_id(0) == 0)
    def _prep():
        mb = means_ref[...]
        abb = alpha_ref[...].astype(bf16)
        mbb = mb.astype(bf16)
        am = jnp.dot(abb, mbb, preferred_element_type=f32)
        atm = jax.lax.dot_general(abb, mbb, (((0,), (0,)), ((), ())),
                                  preferred_element_type=f32)
        abb_ref[...] = abb
        t4_ref[...] = jnp.sum(mb * am, axis=0, keepdims=True)
        m2_ref[...] = (am + atm).astype(bf16)

    x = x_ref[...]
    xb = x.astype(bf16)
    term23 = jnp.dot(xb, m2_ref[...], preferred_element_type=f32)
    xa = jnp.dot(xb, abb_ref[...], preferred_element_type=f32)
    term1 = jnp.sum(xa * x, axis=1, keepdims=True)
    qmin = term1 + jnp.min(t4_ref[...] - term23, axis=1, keepdims=True)
    packed = jnp.broadcast_to(qmin, (qmin.shape[0], _LANE))
    out_ref[...] = packed.T[:1, :]


def kernel(x, means, alpha):
    n, d = x.shape
    d_m, c = means.shape
    assert d == d_m and alpha.shape == (d, d)

    f32 = jnp.float32
    x = x.astype(f32)
    means = means.astype(f32)
    alpha = alpha.astype(f32)

    tn = min(_TN, _round_up(n, _LANE))
    n_pad = _round_up(n, tn)
    num_tiles = n_pad // tn
    x_p = x if n_pad == n else jnp.zeros((n_pad, d), f32).at[:n, :].set(x)

    out = pl.pallas_call(
        _maha_kernel,
        out_shape=jax.ShapeDtypeStruct((num_tiles, 1, tn), f32),
        grid=(num_tiles,),
        in_specs=[
            pl.BlockSpec((tn, d), lambda i: (i, 0)),
            pl.BlockSpec((d, c), lambda i: (0, 0),
                         pipeline_mode=pl.Buffered(1)),
            pl.BlockSpec((d, d), lambda i: (0, 0),
                         pipeline_mode=pl.Buffered(1)),
        ],
        out_specs=pl.BlockSpec((None, 1, tn), lambda i: (i, 0, 0)),
        scratch_shapes=[
            pltpu.VMEM((d, d), jnp.bfloat16),
            pltpu.VMEM((d, c), jnp.bfloat16),
            pltpu.VMEM((1, c), f32),
        ],
        compiler_params=pltpu.CompilerParams(
            dimension_semantics=("arbitrary",),
            vmem_limit_bytes=56 << 20,
        ),
    )(x_p, means, alpha)

    return out.reshape(n_pad)[:n]
